# baseline (speedup 1.0000x reference)
import functools

import jax
import jax.numpy as jnp
from jax import lax
from jax.experimental import pallas as pl
from jax.experimental.pallas import tpu as pltpu
from jax.experimental.pallas import tpu_sc as plsc

N = 10000
E = 320000
D_IN = 128
H = 256
HALF = 128
G = 64
D_OUT_ = 128

NS = 16
EPT = E // NS
CW = 40
NCH = EPT // CW
IB = 25
NIB = NCH // IB
NPT = 624
TAIL_OFF = NS * NPT
TAIL = N - TAIL_OFF

NBUF = 6
BN = 1000
NB = N // BN

_f32 = jnp.float32


def _sc_mesh():
  return plsc.VectorSubcoreMesh(core_axis_name="c", subcore_axis_name="s")


def _deg_body(dst_hbm, zeros_hbm, ones_hbm, out_hbm, idx_v, ones_v, acc_sh):
  c = lax.axis_index("c")
  s = lax.axis_index("s")
  off = s * NPT
  pltpu.sync_copy(zeros_hbm.at[pl.ds(off, NPT)], acc_sh.at[pl.ds(off, NPT)])

  @pl.when(s == NS - 1)
  def _():
    pltpu.sync_copy(zeros_hbm.at[pl.ds(TAIL_OFF, TAIL)],
                    acc_sh.at[pl.ds(TAIL_OFF, TAIL)])

  pltpu.sync_copy(ones_hbm, ones_v)
  plsc.subcore_barrier()

  @pl.loop(0, NIB)
  def _(blk):
    pltpu.sync_copy(dst_hbm.at[c, s, blk], idx_v)

    @pl.loop(0, IB)
    def _(k):
      pltpu.sync_copy(ones_v, acc_sh.at[idx_v.at[k]], add=True)

  plsc.subcore_barrier()
  pltpu.sync_copy(acc_sh.at[pl.ds(off, NPT)], out_hbm.at[c, pl.ds(off, NPT)])

  @pl.when(s == NS - 1)
  def _():
    pltpu.sync_copy(acc_sh.at[pl.ds(TAIL_OFF, TAIL)],
                    out_hbm.at[c, pl.ds(TAIL_OFF, TAIL)])


def _sc_degrees(dst_b):
  zeros = jnp.zeros((N, HALF), _f32)
  ones = jnp.ones((CW, HALF), _f32)
  k = pl.kernel(
      _deg_body,
      out_type=jax.ShapeDtypeStruct((2, N, HALF), _f32),
      mesh=_sc_mesh(),
      scratch_types=[
          pltpu.VMEM((IB, CW), jnp.int32),
          pltpu.VMEM((CW, HALF), _f32),
          pltpu.VMEM_SHARED((N, HALF), _f32),
      ],
  )
  return k(dst_b, zeros, ones)


def _agg_body(h_hbm, src_hbm, dst_hbm, out_hbm, src_v, dst_v, rows_v, acc_sh,
              sem_g, sem_s, sem_i):
  c = lax.axis_index("c")
  s = lax.axis_index("s")
  off = s * NPT

  def gather_start(ib, k, buf):
    pltpu.async_copy(h_hbm.at[src_v.at[ib, k]], rows_v.at[buf],
                     sem_g.at[buf])

  def gather_wait(ib, k, buf):
    pltpu.make_async_copy(h_hbm.at[src_v.at[ib, k]], rows_v.at[buf],
                          sem_g.at[buf]).wait()

  def scatter_start(ib, k, buf):
    pltpu.async_copy(rows_v.at[buf], acc_sh.at[dst_v.at[ib, k]], sem_s,
                     add=True)

  def scatter_wait(ib, k, buf):
    pltpu.make_async_copy(rows_v.at[buf], acc_sh.at[dst_v.at[ib, k]],
                          sem_s).wait()

  for p in range(2):
    j = c * 2 + p
    base = j * N
    pltpu.sync_copy(h_hbm.at[pl.ds(base + off, NPT)],
                    acc_sh.at[pl.ds(off, NPT)])

    @pl.when(s == NS - 1)
    def _():
      pltpu.sync_copy(h_hbm.at[pl.ds(base + TAIL_OFF, TAIL)],
                      acc_sh.at[pl.ds(TAIL_OFF, TAIL)])

    plsc.subcore_barrier()
    pltpu.sync_copy(src_hbm.at[j, s, 0], src_v.at[0])
    pltpu.sync_copy(dst_hbm.at[c, s, 0], dst_v.at[0])
    gather_start(0, 0, 0)

    @pl.loop(0, NIB)
    def _(blk):
      ib = lax.rem(blk, 2)

      @pl.when(blk + 1 < NIB)
      def _():
        pltpu.async_copy(src_hbm.at[j, s, blk + 1], src_v.at[1 - ib], sem_i)
        pltpu.async_copy(dst_hbm.at[c, s, blk + 1], dst_v.at[1 - ib], sem_i)

      @pl.loop(0, IB)
      def _(k):
        buf = lax.rem(k, NBUF)
        nxt = lax.rem(k + 1, NBUF)

        @pl.when(k >= NBUF - 1)
        def _():
          scatter_wait(ib, k - (NBUF - 1), lax.rem(k + 1, NBUF))

        @pl.when(k + 1 < IB)
        def _():
          gather_start(ib, k + 1, nxt)

        gather_wait(ib, k, buf)
        scatter_start(ib, k, buf)

      for t in range(NBUF - 1):
        scatter_wait(ib, IB - (NBUF - 1) + t, (IB - (NBUF - 1) + t) % NBUF)

      @pl.when(blk + 1 < NIB)
      def _():
        pltpu.make_async_copy(src_hbm.at[j, s, 0], src_v.at[1 - ib],
                              sem_i).wait()
        pltpu.make_async_copy(dst_hbm.at[c, s, 0], dst_v.at[1 - ib],
                              sem_i).wait()
        gather_start(1 - ib, 0, 0)

    plsc.subcore_barrier()
    pltpu.sync_copy(acc_sh.at[pl.ds(off, NPT)],
                    out_hbm.at[pl.ds(base + off, NPT)])

    @pl.when(s == NS - 1)
    def _():
      pltpu.sync_copy(acc_sh.at[pl.ds(TAIL_OFF, TAIL)],
                      out_hbm.at[pl.ds(base + TAIL_OFF, TAIL)])


def _sc_aggregate(h_flat, src_jobs, dst_t):
  k = pl.kernel(
      _agg_body,
      out_type=jax.ShapeDtypeStruct((4 * N, HALF), _f32),
      mesh=_sc_mesh(),
      scratch_types=[
          pltpu.VMEM((2, IB, CW), jnp.int32),
          pltpu.VMEM((2, IB, CW), jnp.int32),
          pltpu.VMEM((NBUF, CW, HALF), _f32),
          pltpu.VMEM_SHARED((N, HALF), _f32),
          pltpu.SemaphoreType.DMA((NBUF,)),
          pltpu.SemaphoreType.DMA,
          pltpu.SemaphoreType.DMA,
      ],
  )
  return k(h_flat, src_jobs, dst_t)


def _dinv_of(deg_blk):
  return lax.rsqrt(deg_blk[0, :, 0:1] + 1.0)


def _a1_body(x_ref, w_ref, deg_ref, o_ref):
  x = x_ref[0]
  h = jnp.dot(x, w_ref[...], preferred_element_type=_f32)
  o_ref[0, 0] = h * _dinv_of(deg_ref)


def _tc_conv1(x, w, deg):
  grid = (2, 2, NB)
  return pl.pallas_call(
      _a1_body,
      grid=grid,
      in_specs=[
          pl.BlockSpec((1, BN, D_IN), lambda n, h, i: (n, i, 0)),
          pl.BlockSpec((D_IN, HALF), lambda n, h, i: (0, h)),
          pl.BlockSpec((1, BN, HALF), lambda n, h, i: (n, i, 0)),
      ],
      out_specs=pl.BlockSpec((1, 1, BN, HALF), lambda n, h, i: (n, h, i, 0)),
      out_shape=jax.ShapeDtypeStruct((2, 2, N, HALF), _f32),
  )(x, w, deg)


def _epi_body(agg_ref, deg_ref, b_ref, act_ref, st_ref):
  y = agg_ref[0, 0] * _dinv_of(deg_ref) + b_ref[0, 0]
  y = jnp.maximum(y, 0.0)
  act_ref[0, 0] = y
  sums = jnp.sum(y, axis=0, keepdims=True)
  sq = jnp.sum(y * y, axis=0, keepdims=True)
  upd = jnp.concatenate([sums, sq], axis=0)
  i = pl.program_id(2)

  @pl.when(i == 0)
  def _():
    st_ref[0, 0] = upd

  @pl.when(i > 0)
  def _():
    st_ref[0, 0] = st_ref[0, 0] + upd


def _tc_epilogue(agg, deg, bias_h):
  grid = (2, 2, NB)
  return pl.pallas_call(
      _epi_body,
      grid=grid,
      in_specs=[
          pl.BlockSpec((1, 1, BN, HALF), lambda n, h, i: (n, h, i, 0)),
          pl.BlockSpec((1, BN, HALF), lambda n, h, i: (n, i, 0)),
          pl.BlockSpec((1, 1, HALF), lambda n, h, i: (h, 0, 0)),
      ],
      out_specs=[
          pl.BlockSpec((1, 1, BN, HALF), lambda n, h, i: (n, h, i, 0)),
          pl.BlockSpec((1, 1, 2, HALF), lambda n, h, i: (n, h, 0, 0)),
      ],
      out_shape=[
          jax.ShapeDtypeStruct((2, 2, N, HALF), _f32),
          jax.ShapeDtypeStruct((2, 2, 2, HALF), _f32),
      ],
  )(agg, deg, bias_h)


def _bn_half(act, st, g, be):
  m = st[0:1, :] * (1.0 / N)
  v = st[1:2, :] * (1.0 / N) - m * m
  a = g * lax.rsqrt(v + 1e-5)
  c = be - m * a
  return act * a + c


def _amat_body(alo_ref, ahi_ref, stlo_ref, sthi_ref, glo_ref, ghi_ref,
               belo_ref, behi_ref, wlo_ref, whi_ref, deg_ref, o_ref):
  x0 = _bn_half(alo_ref[0, 0], stlo_ref[0, 0], glo_ref[0], belo_ref[0])
  x1 = _bn_half(ahi_ref[0, 0], sthi_ref[0, 0], ghi_ref[0], behi_ref[0])
  h = (jnp.dot(x0, wlo_ref[0], preferred_element_type=_f32) +
       jnp.dot(x1, whi_ref[0], preferred_element_type=_f32))
  o_ref[0, 0] = h * _dinv_of(deg_ref)


def _tc_bn_matmul(act, st, g_h, be_h, w_r, deg):
  grid = (2, 2, NB)
  act_spec = lambda k: pl.BlockSpec((1, 1, BN, HALF),
                                    lambda n, h, i, _k=k: (n, _k, i, 0))
  st_spec = lambda k: pl.BlockSpec((1, 1, 2, HALF),
                                   lambda n, h, i, _k=k: (n, _k, 0, 0))
  vec_spec = lambda k: pl.BlockSpec((1, 1, HALF),
                                    lambda n, h, i, _k=k: (_k, 0, 0))
  w_spec = lambda k: pl.BlockSpec((1, HALF, HALF),
                                  lambda n, h, i, _k=k: (_k, 0, h))
  return pl.pallas_call(
      _amat_body,
      grid=grid,
      in_specs=[
          act_spec(0), act_spec(1), st_spec(0), st_spec(1),
          vec_spec(0), vec_spec(1), vec_spec(0), vec_spec(1),
          w_spec(0), w_spec(1),
          pl.BlockSpec((1, BN, HALF), lambda n, h, i: (n, i, 0)),
      ],
      out_specs=pl.BlockSpec((1, 1, BN, HALF), lambda n, h, i: (n, h, i, 0)),
      out_shape=jax.ShapeDtypeStruct((2, 2, N, HALF), _f32),
  )(act, act, st, st, g_h, g_h, be_h, be_h, w_r, w_r, deg)


def _pool_body(agg_ref, deg_ref, b_ref, batch_ref, pooled_ref, cnt_ref):
  y = agg_ref[0, 0] * _dinv_of(deg_ref) + b_ref[0, 0]
  bidx = batch_ref[0, 0]
  p1h = (lax.broadcasted_iota(jnp.int32, (G, BN), 0) == bidx).astype(_f32)
  contrib = jnp.dot(p1h, y, preferred_element_type=_f32)
  cnt = jnp.dot(p1h, jnp.ones((BN, HALF), _f32), preferred_element_type=_f32)
  h = pl.program_id(1)
  i = pl.program_id(2)

  @pl.when(i == 0)
  def _():
    pooled_ref[0, 0] = contrib

  @pl.when(i > 0)
  def _():
    pooled_ref[0, 0] = pooled_ref[0, 0] + contrib

  @pl.when((h == 0) & (i == 0))
  def _():
    cnt_ref[0] = cnt

  @pl.when((h == 0) & (i > 0))
  def _():
    cnt_ref[0] = cnt_ref[0] + cnt


def _tc_pool(agg, deg, bias_h, batch_r):
  grid = (2, 2, NB)
  return pl.pallas_call(
      _pool_body,
      grid=grid,
      in_specs=[
          pl.BlockSpec((1, 1, BN, HALF), lambda n, h, i: (n, h, i, 0)),
          pl.BlockSpec((1, BN, HALF), lambda n, h, i: (n, i, 0)),
          pl.BlockSpec((1, 1, HALF), lambda n, h, i: (h, 0, 0)),
          pl.BlockSpec((1, 1, 1, BN), lambda n, h, i: (n, i, 0, 0)),
      ],
      out_specs=[
          pl.BlockSpec((1, 1, G, HALF), lambda n, h, i: (n, h, 0, 0)),
          pl.BlockSpec((1, G, HALF), lambda n, h, i: (n, 0, 0)),
      ],
      out_shape=[
          jax.ShapeDtypeStruct((2, 2, G, HALF), _f32),
          jax.ShapeDtypeStruct((2, G, HALF), _f32),
      ],
  )(agg, deg, bias_h, batch_r)


def _head_body(pooled_ref, cnt_ref, wl1_ref, bl1_ref, wl2_ref, bl2_ref,
               o_ref):
  def embed(n):
    p = jnp.concatenate([pooled_ref[n, 0], pooled_ref[n, 1]], axis=1)
    c = jnp.maximum(cnt_ref[n][:, 0:1], 1.0)
    m = p / c
    t = jnp.dot(m, wl1_ref[...], preferred_element_type=_f32) + bl1_ref[...]
    t = jnp.maximum(t, 0.0)
    e = jnp.dot(t, wl2_ref[...], preferred_element_type=_f32) + bl2_ref[...]
    return jnp.maximum(e, 0.0)

  e1 = embed(0)
  e2 = embed(1)
  o_ref[...] = jnp.sum(jnp.abs(e1 - e2), axis=1, keepdims=True)


def _tc_head(pooled, cnt, wl1, bl1, wl2, bl2):
  return pl.pallas_call(
      _head_body,
      out_shape=jax.ShapeDtypeStruct((G, 1), _f32),
  )(pooled, cnt, wl1, bl1, wl2, bl2)


def kernel(x1, edge_index1, batch1, x2, edge_index2, batch2,
           Wc1, bc1, Wc2, bc2, Wc3, bc3, g1, be1, g2, be2, Wl1, bl1, Wl2,
           bl2):
  x = jnp.stack([x1, x2])
  src = jnp.stack([edge_index1[0], edge_index2[0]])
  dst = jnp.stack([edge_index1[1], edge_index2[1]])

  joff = jnp.arange(4, dtype=jnp.int32)[:, None] * N
  src_jobs = (jnp.repeat(src, 2, axis=0) + joff).reshape(4, NS, NIB, IB, CW)
  dst_t = dst.reshape(2, NS, NCH, CW)
  dst_b = dst.reshape(2, NS, NIB, IB, CW)
  batch_r = jnp.stack([batch1, batch2]).reshape(2, NB, 1, BN)

  bc1h = bc1.reshape(2, 1, HALF)
  bc2h = bc2.reshape(2, 1, HALF)
  bc3h = bc3.reshape(2, 1, HALF)
  g1h = g1.reshape(2, 1, HALF)
  be1h = be1.reshape(2, 1, HALF)
  g2h = g2.reshape(2, 1, HALF)
  be2h = be2.reshape(2, 1, HALF)
  wc2r = Wc2.reshape(2, HALF, H)
  wc3r = Wc3.reshape(2, HALF, H)

  deg = _sc_degrees(dst_b)

  h1 = _tc_conv1(x, Wc1, deg)
  agg1 = _sc_aggregate(h1.reshape(4 * N, HALF), src_jobs, dst_b)
  act1, st1 = _tc_epilogue(agg1.reshape(2, 2, N, HALF), deg, bc1h)

  h2 = _tc_bn_matmul(act1, st1, g1h, be1h, wc2r, deg)
  agg2 = _sc_aggregate(h2.reshape(4 * N, HALF), src_jobs, dst_b)
  act2, st2 = _tc_epilogue(agg2.reshape(2, 2, N, HALF), deg, bc2h)

  h3 = _tc_bn_matmul(act2, st2, g2h, be2h, wc3r, deg)
  agg3 = _sc_aggregate(h3.reshape(4 * N, HALF), src_jobs, dst_b)
  pooled, cnt = _tc_pool(agg3.reshape(2, 2, N, HALF), deg, bc3h, batch_r)

  out = _tc_head(pooled, cnt, Wl1, bl1.reshape(1, H), Wl2,
                 bl2.reshape(1, D_OUT_))
  return out[:, 0]


# speedup vs baseline: 1.3402x; 1.3402x over previous
import functools

import jax
import jax.numpy as jnp
from jax import lax
from jax.experimental import pallas as pl
from jax.experimental.pallas import tpu as pltpu
from jax.experimental.pallas import tpu_sc as plsc

N = 10000
E = 320000
D_IN = 128
H = 256
HALF = 128
G = 64
D_OUT_ = 128

NS = 16
EPT = E // NS
CW = 80
NCH = EPT // CW
IB = 25
NIB = NCH // IB
NPT = 624
TAIL_OFF = NS * NPT
TAIL = N - TAIL_OFF

NBUF = 3
BN = 1000
NB = N // BN

_f32 = jnp.float32


def _sc_mesh():
  return plsc.VectorSubcoreMesh(core_axis_name="c", subcore_axis_name="s")


def _deg_body(dst_hbm, zeros_hbm, ones_hbm, out_hbm, idx_v, ones_v, acc_sh):
  c = lax.axis_index("c")
  s = lax.axis_index("s")
  off = s * NPT
  pltpu.sync_copy(zeros_hbm.at[pl.ds(off, NPT)], acc_sh.at[pl.ds(off, NPT)])

  @pl.when(s == NS - 1)
  def _():
    pltpu.sync_copy(zeros_hbm.at[pl.ds(TAIL_OFF, TAIL)],
                    acc_sh.at[pl.ds(TAIL_OFF, TAIL)])

  pltpu.sync_copy(ones_hbm, ones_v)
  plsc.subcore_barrier()

  @pl.loop(0, NIB)
  def _(blk):
    pltpu.sync_copy(dst_hbm.at[c, s, blk], idx_v)

    @pl.loop(0, IB)
    def _(k):
      pltpu.sync_copy(ones_v, acc_sh.at[idx_v.at[k]], add=True)

  plsc.subcore_barrier()
  pltpu.sync_copy(acc_sh.at[pl.ds(off, NPT)], out_hbm.at[c, pl.ds(off, NPT)])

  @pl.when(s == NS - 1)
  def _():
    pltpu.sync_copy(acc_sh.at[pl.ds(TAIL_OFF, TAIL)],
                    out_hbm.at[c, pl.ds(TAIL_OFF, TAIL)])


def _sc_degrees(dst_b):
  zeros = jnp.zeros((N, HALF), _f32)
  ones = jnp.ones((CW, HALF), _f32)
  k = pl.kernel(
      _deg_body,
      out_type=jax.ShapeDtypeStruct((2, N, HALF), _f32),
      mesh=_sc_mesh(),
      scratch_types=[
          pltpu.VMEM((IB, CW), jnp.int32),
          pltpu.VMEM((CW, HALF), _f32),
          pltpu.VMEM_SHARED((N, HALF), _f32),
      ],
  )
  return k(dst_b, zeros, ones)


def _agg_body(h_hbm, src_hbm, dst_hbm, out_hbm, src_v, dst_v, rows_v, acc_sh,
              sem_g, sem_s, sem_i):
  c = lax.axis_index("c")
  s = lax.axis_index("s")
  off = s * NPT

  def gather_start(ib, k, buf):
    pltpu.async_copy(h_hbm.at[src_v.at[ib, k]], rows_v.at[buf],
                     sem_g.at[buf])

  def gather_wait(ib, k, buf):
    pltpu.make_async_copy(h_hbm.at[src_v.at[ib, k]], rows_v.at[buf],
                          sem_g.at[buf]).wait()

  def scatter_start(ib, k, buf):
    pltpu.async_copy(rows_v.at[buf], acc_sh.at[dst_v.at[ib, k]], sem_s,
                     add=True)

  def scatter_wait(ib, k, buf):
    pltpu.make_async_copy(rows_v.at[buf], acc_sh.at[dst_v.at[ib, k]],
                          sem_s).wait()

  for p in range(2):
    j = c * 2 + p
    base = j * N
    pltpu.sync_copy(h_hbm.at[pl.ds(base + off, NPT)],
                    acc_sh.at[pl.ds(off, NPT)])

    @pl.when(s == NS - 1)
    def _():
      pltpu.sync_copy(h_hbm.at[pl.ds(base + TAIL_OFF, TAIL)],
                      acc_sh.at[pl.ds(TAIL_OFF, TAIL)])

    plsc.subcore_barrier()
    pltpu.sync_copy(src_hbm.at[j, s, 0], src_v.at[0])
    pltpu.sync_copy(dst_hbm.at[c, s, 0], dst_v.at[0])
    gather_start(0, 0, 0)

    @pl.loop(0, NIB)
    def _(blk):
      ib = lax.rem(blk, 2)

      @pl.when(blk + 1 < NIB)
      def _():
        pltpu.async_copy(src_hbm.at[j, s, blk + 1], src_v.at[1 - ib], sem_i)
        pltpu.async_copy(dst_hbm.at[c, s, blk + 1], dst_v.at[1 - ib], sem_i)

      @pl.loop(0, IB)
      def _(k):
        buf = lax.rem(k, NBUF)
        nxt = lax.rem(k + 1, NBUF)

        @pl.when(k >= NBUF - 1)
        def _():
          scatter_wait(ib, k - (NBUF - 1), lax.rem(k + 1, NBUF))

        @pl.when(k + 1 < IB)
        def _():
          gather_start(ib, k + 1, nxt)

        gather_wait(ib, k, buf)
        scatter_start(ib, k, buf)

      for t in range(NBUF - 1):
        scatter_wait(ib, IB - (NBUF - 1) + t, (IB - (NBUF - 1) + t) % NBUF)

      @pl.when(blk + 1 < NIB)
      def _():
        pltpu.make_async_copy(src_hbm.at[j, s, 0], src_v.at[1 - ib],
                              sem_i).wait()
        pltpu.make_async_copy(dst_hbm.at[c, s, 0], dst_v.at[1 - ib],
                              sem_i).wait()
        gather_start(1 - ib, 0, 0)

    plsc.subcore_barrier()
    pltpu.sync_copy(acc_sh.at[pl.ds(off, NPT)],
                    out_hbm.at[pl.ds(base + off, NPT)])

    @pl.when(s == NS - 1)
    def _():
      pltpu.sync_copy(acc_sh.at[pl.ds(TAIL_OFF, TAIL)],
                      out_hbm.at[pl.ds(base + TAIL_OFF, TAIL)])


def _sc_aggregate(h_flat, src_jobs, dst_t):
  k = pl.kernel(
      _agg_body,
      out_type=jax.ShapeDtypeStruct((4 * N, HALF), _f32),
      mesh=_sc_mesh(),
      scratch_types=[
          pltpu.VMEM((2, IB, CW), jnp.int32),
          pltpu.VMEM((2, IB, CW), jnp.int32),
          pltpu.VMEM((NBUF, CW, HALF), _f32),
          pltpu.VMEM_SHARED((N, HALF), _f32),
          pltpu.SemaphoreType.DMA((NBUF,)),
          pltpu.SemaphoreType.DMA,
          pltpu.SemaphoreType.DMA,
      ],
  )
  return k(h_flat, src_jobs, dst_t)


def _dinv_of(deg_blk):
  return lax.rsqrt(deg_blk[0, :, 0:1] + 1.0)


def _a1_body(x_ref, w_ref, deg_ref, o_ref):
  x = x_ref[0]
  h = jnp.dot(x, w_ref[...], preferred_element_type=_f32)
  o_ref[0, 0] = h * _dinv_of(deg_ref)


def _tc_conv1(x, w, deg):
  grid = (2, 2, NB)
  return pl.pallas_call(
      _a1_body,
      grid=grid,
      in_specs=[
          pl.BlockSpec((1, BN, D_IN), lambda n, h, i: (n, i, 0)),
          pl.BlockSpec((D_IN, HALF), lambda n, h, i: (0, h)),
          pl.BlockSpec((1, BN, HALF), lambda n, h, i: (n, i, 0)),
      ],
      out_specs=pl.BlockSpec((1, 1, BN, HALF), lambda n, h, i: (n, h, i, 0)),
      out_shape=jax.ShapeDtypeStruct((2, 2, N, HALF), _f32),
  )(x, w, deg)


def _epi_body(agg_ref, deg_ref, b_ref, act_ref, st_ref):
  y = agg_ref[0, 0] * _dinv_of(deg_ref) + b_ref[0, 0]
  y = jnp.maximum(y, 0.0)
  act_ref[0, 0] = y
  sums = jnp.sum(y, axis=0, keepdims=True)
  sq = jnp.sum(y * y, axis=0, keepdims=True)
  upd = jnp.concatenate([sums, sq], axis=0)
  i = pl.program_id(2)

  @pl.when(i == 0)
  def _():
    st_ref[0, 0] = upd

  @pl.when(i > 0)
  def _():
    st_ref[0, 0] = st_ref[0, 0] + upd


def _tc_epilogue(agg, deg, bias_h):
  grid = (2, 2, NB)
  return pl.pallas_call(
      _epi_body,
      grid=grid,
      in_specs=[
          pl.BlockSpec((1, 1, BN, HALF), lambda n, h, i: (n, h, i, 0)),
          pl.BlockSpec((1, BN, HALF), lambda n, h, i: (n, i, 0)),
          pl.BlockSpec((1, 1, HALF), lambda n, h, i: (h, 0, 0)),
      ],
      out_specs=[
          pl.BlockSpec((1, 1, BN, HALF), lambda n, h, i: (n, h, i, 0)),
          pl.BlockSpec((1, 1, 2, HALF), lambda n, h, i: (n, h, 0, 0)),
      ],
      out_shape=[
          jax.ShapeDtypeStruct((2, 2, N, HALF), _f32),
          jax.ShapeDtypeStruct((2, 2, 2, HALF), _f32),
      ],
  )(agg, deg, bias_h)


def _bn_half(act, st, g, be):
  m = st[0:1, :] * (1.0 / N)
  v = st[1:2, :] * (1.0 / N) - m * m
  a = g * lax.rsqrt(v + 1e-5)
  c = be - m * a
  return act * a + c


def _amat_body(alo_ref, ahi_ref, stlo_ref, sthi_ref, glo_ref, ghi_ref,
               belo_ref, behi_ref, wlo_ref, whi_ref, deg_ref, o_ref):
  x0 = _bn_half(alo_ref[0, 0], stlo_ref[0, 0], glo_ref[0], belo_ref[0])
  x1 = _bn_half(ahi_ref[0, 0], sthi_ref[0, 0], ghi_ref[0], behi_ref[0])
  h = (jnp.dot(x0, wlo_ref[0], preferred_element_type=_f32) +
       jnp.dot(x1, whi_ref[0], preferred_element_type=_f32))
  o_ref[0, 0] = h * _dinv_of(deg_ref)


def _tc_bn_matmul(act, st, g_h, be_h, w_r, deg):
  grid = (2, 2, NB)
  act_spec = lambda k: pl.BlockSpec((1, 1, BN, HALF),
                                    lambda n, h, i, _k=k: (n, _k, i, 0))
  st_spec = lambda k: pl.BlockSpec((1, 1, 2, HALF),
                                   lambda n, h, i, _k=k: (n, _k, 0, 0))
  vec_spec = lambda k: pl.BlockSpec((1, 1, HALF),
                                    lambda n, h, i, _k=k: (_k, 0, 0))
  w_spec = lambda k: pl.BlockSpec((1, HALF, HALF),
                                  lambda n, h, i, _k=k: (_k, 0, h))
  return pl.pallas_call(
      _amat_body,
      grid=grid,
      in_specs=[
          act_spec(0), act_spec(1), st_spec(0), st_spec(1),
          vec_spec(0), vec_spec(1), vec_spec(0), vec_spec(1),
          w_spec(0), w_spec(1),
          pl.BlockSpec((1, BN, HALF), lambda n, h, i: (n, i, 0)),
      ],
      out_specs=pl.BlockSpec((1, 1, BN, HALF), lambda n, h, i: (n, h, i, 0)),
      out_shape=jax.ShapeDtypeStruct((2, 2, N, HALF), _f32),
  )(act, act, st, st, g_h, g_h, be_h, be_h, w_r, w_r, deg)


def _pool_body(agg_ref, deg_ref, b_ref, batch_ref, pooled_ref, cnt_ref):
  y = agg_ref[0, 0] * _dinv_of(deg_ref) + b_ref[0, 0]
  bidx = batch_ref[0, 0]
  p1h = (lax.broadcasted_iota(jnp.int32, (G, BN), 0) == bidx).astype(_f32)
  contrib = jnp.dot(p1h, y, preferred_element_type=_f32)
  cnt = jnp.dot(p1h, jnp.ones((BN, HALF), _f32), preferred_element_type=_f32)
  h = pl.program_id(1)
  i = pl.program_id(2)

  @pl.when(i == 0)
  def _():
    pooled_ref[0, 0] = contrib

  @pl.when(i > 0)
  def _():
    pooled_ref[0, 0] = pooled_ref[0, 0] + contrib

  @pl.when((h == 0) & (i == 0))
  def _():
    cnt_ref[0] = cnt

  @pl.when((h == 0) & (i > 0))
  def _():
    cnt_ref[0] = cnt_ref[0] + cnt


def _tc_pool(agg, deg, bias_h, batch_r):
  grid = (2, 2, NB)
  return pl.pallas_call(
      _pool_body,
      grid=grid,
      in_specs=[
          pl.BlockSpec((1, 1, BN, HALF), lambda n, h, i: (n, h, i, 0)),
          pl.BlockSpec((1, BN, HALF), lambda n, h, i: (n, i, 0)),
          pl.BlockSpec((1, 1, HALF), lambda n, h, i: (h, 0, 0)),
          pl.BlockSpec((1, 1, 1, BN), lambda n, h, i: (n, i, 0, 0)),
      ],
      out_specs=[
          pl.BlockSpec((1, 1, G, HALF), lambda n, h, i: (n, h, 0, 0)),
          pl.BlockSpec((1, G, HALF), lambda n, h, i: (n, 0, 0)),
      ],
      out_shape=[
          jax.ShapeDtypeStruct((2, 2, G, HALF), _f32),
          jax.ShapeDtypeStruct((2, G, HALF), _f32),
      ],
  )(agg, deg, bias_h, batch_r)


def _head_body(pooled_ref, cnt_ref, wl1_ref, bl1_ref, wl2_ref, bl2_ref,
               o_ref):
  def embed(n):
    p = jnp.concatenate([pooled_ref[n, 0], pooled_ref[n, 1]], axis=1)
    c = jnp.maximum(cnt_ref[n][:, 0:1], 1.0)
    m = p / c
    t = jnp.dot(m, wl1_ref[...], preferred_element_type=_f32) + bl1_ref[...]
    t = jnp.maximum(t, 0.0)
    e = jnp.dot(t, wl2_ref[...], preferred_element_type=_f32) + bl2_ref[...]
    return jnp.maximum(e, 0.0)

  e1 = embed(0)
  e2 = embed(1)
  o_ref[...] = jnp.sum(jnp.abs(e1 - e2), axis=1, keepdims=True)


def _tc_head(pooled, cnt, wl1, bl1, wl2, bl2):
  return pl.pallas_call(
      _head_body,
      out_shape=jax.ShapeDtypeStruct((G, 1), _f32),
  )(pooled, cnt, wl1, bl1, wl2, bl2)


def kernel(x1, edge_index1, batch1, x2, edge_index2, batch2,
           Wc1, bc1, Wc2, bc2, Wc3, bc3, g1, be1, g2, be2, Wl1, bl1, Wl2,
           bl2):
  x = jnp.stack([x1, x2])
  src = jnp.stack([edge_index1[0], edge_index2[0]])
  dst = jnp.stack([edge_index1[1], edge_index2[1]])

  joff = jnp.arange(4, dtype=jnp.int32)[:, None] * N
  src_jobs = (jnp.repeat(src, 2, axis=0) + joff).reshape(4, NS, NIB, IB, CW)
  dst_t = dst.reshape(2, NS, NCH, CW)
  dst_b = dst.reshape(2, NS, NIB, IB, CW)
  batch_r = jnp.stack([batch1, batch2]).reshape(2, NB, 1, BN)

  bc1h = bc1.reshape(2, 1, HALF)
  bc2h = bc2.reshape(2, 1, HALF)
  bc3h = bc3.reshape(2, 1, HALF)
  g1h = g1.reshape(2, 1, HALF)
  be1h = be1.reshape(2, 1, HALF)
  g2h = g2.reshape(2, 1, HALF)
  be2h = be2.reshape(2, 1, HALF)
  wc2r = Wc2.reshape(2, HALF, H)
  wc3r = Wc3.reshape(2, HALF, H)

  deg = _sc_degrees(dst_b)

  h1 = _tc_conv1(x, Wc1, deg)
  agg1 = _sc_aggregate(h1.reshape(4 * N, HALF), src_jobs, dst_b)
  act1, st1 = _tc_epilogue(agg1.reshape(2, 2, N, HALF), deg, bc1h)

  h2 = _tc_bn_matmul(act1, st1, g1h, be1h, wc2r, deg)
  agg2 = _sc_aggregate(h2.reshape(4 * N, HALF), src_jobs, dst_b)
  act2, st2 = _tc_epilogue(agg2.reshape(2, 2, N, HALF), deg, bc2h)

  h3 = _tc_bn_matmul(act2, st2, g2h, be2h, wc3r, deg)
  agg3 = _sc_aggregate(h3.reshape(4 * N, HALF), src_jobs, dst_b)
  pooled, cnt = _tc_pool(agg3.reshape(2, 2, N, HALF), deg, bc3h, batch_r)

  out = _tc_head(pooled, cnt, Wl1, bl1.reshape(1, H), Wl2,
                 bl2.reshape(1, D_OUT_))
  return out[:, 0]


# speedup vs baseline: 1.4486x; 1.0808x over previous
import jax
import jax.numpy as jnp
from jax import lax
from jax.experimental import pallas as pl
from jax.experimental.pallas import tpu as pltpu
from jax.experimental.pallas import tpu_sc as plsc

N = 10000
E = 320000
D_IN = 128
H = 256
HALF = 128
G = 64
D_OUT_ = 128

NS = 16
EPT = E // NS
CW = 80
NCH = EPT // CW
IB = 25
NIB = NCH // IB
NIBD = NIB // 2
NPT = 624
TAIL_OFF = NS * NPT
TAIL = N - TAIL_OFF

NBUF = 3
BN = 1000
NB = N // BN

_f32 = jnp.float32


def _sc_mesh():
  return plsc.VectorSubcoreMesh(core_axis_name="c", subcore_axis_name="s")


def _deg_body(dst_hbm, zeros_hbm, ones_hbm, out_hbm, idx_v, ones_v, acc_sh):
  c = lax.axis_index("c")
  s = lax.axis_index("s")
  off = s * NPT
  pltpu.sync_copy(zeros_hbm.at[pl.ds(off, NPT)], acc_sh.at[pl.ds(off, NPT)])

  @pl.when(s == NS - 1)
  def _():
    pltpu.sync_copy(zeros_hbm.at[pl.ds(TAIL_OFF, TAIL)],
                    acc_sh.at[pl.ds(TAIL_OFF, TAIL)])

  pltpu.sync_copy(ones_hbm, ones_v)
  plsc.subcore_barrier()

  @pl.loop(0, NIBD)
  def _(blk):
    pltpu.sync_copy(dst_hbm.at[c, s, blk], idx_v)

    @pl.loop(0, IB)
    def _(k):
      pltpu.sync_copy(ones_v, acc_sh.at[idx_v.at[k]], add=True)

  plsc.subcore_barrier()
  pltpu.sync_copy(acc_sh.at[pl.ds(off, NPT)], out_hbm.at[c, pl.ds(off, NPT)])

  @pl.when(s == NS - 1)
  def _():
    pltpu.sync_copy(acc_sh.at[pl.ds(TAIL_OFF, TAIL)],
                    out_hbm.at[c, pl.ds(TAIL_OFF, TAIL)])


def _sc_degrees(dst_d):
  zeros = jnp.zeros((N, HALF), _f32)
  ones = jnp.ones((CW, HALF), _f32)
  k = pl.kernel(
      _deg_body,
      out_type=jax.ShapeDtypeStruct((2, N, HALF), _f32),
      mesh=_sc_mesh(),
      scratch_types=[
          pltpu.VMEM((IB, CW), jnp.int32),
          pltpu.VMEM((CW, HALF), _f32),
          pltpu.VMEM_SHARED((N, HALF), _f32),
      ],
  )
  return k(dst_d, zeros, ones)


def _agg_body(h_hbm, src_hbm, dst_hbm, out_hbm, src_v, dst_v, rows_v, acc_sh,
              sem_g, sem_s, sem_i):
  c = lax.axis_index("c")
  s = lax.axis_index("s")
  off = s * NPT
  base = c * N

  def gather_start(ib, k, buf):
    pltpu.async_copy(h_hbm.at[src_v.at[ib, k]], rows_v.at[buf],
                     sem_g.at[buf])

  def gather_wait(ib, k, buf):
    pltpu.make_async_copy(h_hbm.at[src_v.at[ib, k]], rows_v.at[buf],
                          sem_g.at[buf]).wait()

  def scatter_start(ib, k, buf):
    pltpu.async_copy(rows_v.at[buf], acc_sh.at[dst_v.at[ib, k]], sem_s,
                     add=True)

  def scatter_wait(ib, k, buf):
    pltpu.make_async_copy(rows_v.at[buf], acc_sh.at[dst_v.at[ib, k]],
                          sem_s).wait()

  pltpu.sync_copy(h_hbm.at[pl.ds(base + off, NPT)],
                  acc_sh.at[pl.ds(off, NPT)])

  @pl.when(s == NS - 1)
  def _():
    pltpu.sync_copy(h_hbm.at[pl.ds(base + TAIL_OFF, TAIL)],
                    acc_sh.at[pl.ds(TAIL_OFF, TAIL)])

  plsc.subcore_barrier()
  pltpu.sync_copy(src_hbm.at[c, s, 0], src_v.at[0])
  pltpu.sync_copy(dst_hbm.at[s, 0], dst_v.at[0])
  gather_start(0, 0, 0)

  @pl.loop(0, NIB)
  def _(blk):
    ib = lax.rem(blk, 2)

    @pl.when(blk + 1 < NIB)
    def _():
      pltpu.async_copy(src_hbm.at[c, s, blk + 1], src_v.at[1 - ib], sem_i)
      pltpu.async_copy(dst_hbm.at[s, blk + 1], dst_v.at[1 - ib], sem_i)

    @pl.loop(0, IB)
    def _(k):
      buf = lax.rem(k, NBUF)
      nxt = lax.rem(k + 1, NBUF)

      @pl.when(k >= NBUF - 1)
      def _():
        scatter_wait(ib, k - (NBUF - 1), lax.rem(k + 1, NBUF))

      @pl.when(k + 1 < IB)
      def _():
        gather_start(ib, k + 1, nxt)

      gather_wait(ib, k, buf)
      scatter_start(ib, k, buf)

    for t in range(NBUF - 1):
      scatter_wait(ib, IB - (NBUF - 1) + t, (IB - (NBUF - 1) + t) % NBUF)

    @pl.when(blk + 1 < NIB)
    def _():
      pltpu.make_async_copy(src_hbm.at[c, s, 0], src_v.at[1 - ib],
                            sem_i).wait()
      pltpu.make_async_copy(dst_hbm.at[s, 0], dst_v.at[1 - ib],
                            sem_i).wait()
      gather_start(1 - ib, 0, 0)

  plsc.subcore_barrier()
  pltpu.sync_copy(acc_sh.at[pl.ds(off, NPT)],
                  out_hbm.at[pl.ds(base + off, NPT)])

  @pl.when(s == NS - 1)
  def _():
    pltpu.sync_copy(acc_sh.at[pl.ds(TAIL_OFF, TAIL)],
                    out_hbm.at[pl.ds(base + TAIL_OFF, TAIL)])


def _sc_aggregate(h_flat, src_jobs, dst_b):
  k = pl.kernel(
      _agg_body,
      out_type=jax.ShapeDtypeStruct((2 * N, HALF), _f32),
      mesh=_sc_mesh(),
      scratch_types=[
          pltpu.VMEM((2, IB, CW), jnp.int32),
          pltpu.VMEM((2, IB, CW), jnp.int32),
          pltpu.VMEM((NBUF, CW, HALF), _f32),
          pltpu.VMEM_SHARED((N, HALF), _f32),
          pltpu.SemaphoreType.DMA((NBUF,)),
          pltpu.SemaphoreType.DMA,
          pltpu.SemaphoreType.DMA,
      ],
  )
  return k(h_flat, src_jobs, dst_b)


def _dinv_of(d0_ref, d1_ref):
  return lax.rsqrt(d0_ref[0, :, 0:1] + d1_ref[0, :, 0:1] + 1.0)


def _a1_body(x_ref, w_ref, d0_ref, d1_ref, o_ref):
  h = jnp.dot(x_ref[...], w_ref[...], preferred_element_type=_f32)
  o_ref[0] = h * _dinv_of(d0_ref, d1_ref)


def _tc_conv1(x, w, deg):
  grid = (2, NB)
  return pl.pallas_call(
      _a1_body,
      grid=grid,
      in_specs=[
          pl.BlockSpec((BN, D_IN), lambda h, i: (i, 0)),
          pl.BlockSpec((D_IN, HALF), lambda h, i: (0, h)),
          pl.BlockSpec((1, BN, HALF), lambda h, i: (0, i, 0)),
          pl.BlockSpec((1, BN, HALF), lambda h, i: (1, i, 0)),
      ],
      out_specs=pl.BlockSpec((1, BN, HALF), lambda h, i: (h, i, 0)),
      out_shape=jax.ShapeDtypeStruct((2, N, HALF), _f32),
  )(x, w, deg, deg)


def _epi_body(agg_ref, d0_ref, d1_ref, b_ref, act_ref, st_ref):
  y = agg_ref[0] * _dinv_of(d0_ref, d1_ref) + b_ref[0]
  y = jnp.maximum(y, 0.0)
  act_ref[0] = y
  sums = jnp.sum(y, axis=0, keepdims=True)
  sq = jnp.sum(y * y, axis=0, keepdims=True)
  upd = jnp.concatenate([sums, sq], axis=0)
  i = pl.program_id(1)

  @pl.when(i == 0)
  def _():
    st_ref[0] = upd

  @pl.when(i > 0)
  def _():
    st_ref[0] = st_ref[0] + upd


def _tc_epilogue(agg, deg, bias_h):
  grid = (2, NB)
  return pl.pallas_call(
      _epi_body,
      grid=grid,
      in_specs=[
          pl.BlockSpec((1, BN, HALF), lambda h, i: (h, i, 0)),
          pl.BlockSpec((1, BN, HALF), lambda h, i: (0, i, 0)),
          pl.BlockSpec((1, BN, HALF), lambda h, i: (1, i, 0)),
          pl.BlockSpec((1, 1, HALF), lambda h, i: (h, 0, 0)),
      ],
      out_specs=[
          pl.BlockSpec((1, BN, HALF), lambda h, i: (h, i, 0)),
          pl.BlockSpec((1, 2, HALF), lambda h, i: (h, 0, 0)),
      ],
      out_shape=[
          jax.ShapeDtypeStruct((2, N, HALF), _f32),
          jax.ShapeDtypeStruct((2, 2, HALF), _f32),
      ],
  )(agg, deg, deg, bias_h)


def _bn_half(act, st, g, be):
  m = st[0:1, :] * (1.0 / N)
  v = st[1:2, :] * (1.0 / N) - m * m
  a = g * lax.rsqrt(v + 1e-5)
  c = be - m * a
  return act * a + c


def _amat_body(alo_ref, ahi_ref, stlo_ref, sthi_ref, glo_ref, ghi_ref,
               belo_ref, behi_ref, wlo_ref, whi_ref, d0_ref, d1_ref, o_ref):
  x0 = _bn_half(alo_ref[0], stlo_ref[0], glo_ref[0], belo_ref[0])
  x1 = _bn_half(ahi_ref[0], sthi_ref[0], ghi_ref[0], behi_ref[0])
  h = (jnp.dot(x0, wlo_ref[0], preferred_element_type=_f32) +
       jnp.dot(x1, whi_ref[0], preferred_element_type=_f32))
  o_ref[0] = h * _dinv_of(d0_ref, d1_ref)


def _tc_bn_matmul(act, st, g_h, be_h, w_r, deg):
  grid = (2, NB)
  act_spec = lambda k: pl.BlockSpec((1, BN, HALF),
                                    lambda h, i, _k=k: (_k, i, 0))
  st_spec = lambda k: pl.BlockSpec((1, 2, HALF),
                                   lambda h, i, _k=k: (_k, 0, 0))
  vec_spec = lambda k: pl.BlockSpec((1, 1, HALF),
                                    lambda h, i, _k=k: (_k, 0, 0))
  w_spec = lambda k: pl.BlockSpec((1, HALF, HALF),
                                  lambda h, i, _k=k: (_k, 0, h))
  deg_spec = lambda k: pl.BlockSpec((1, BN, HALF),
                                    lambda h, i, _k=k: (_k, i, 0))
  return pl.pallas_call(
      _amat_body,
      grid=grid,
      in_specs=[
          act_spec(0), act_spec(1), st_spec(0), st_spec(1),
          vec_spec(0), vec_spec(1), vec_spec(0), vec_spec(1),
          w_spec(0), w_spec(1), deg_spec(0), deg_spec(1),
      ],
      out_specs=pl.BlockSpec((1, BN, HALF), lambda h, i: (h, i, 0)),
      out_shape=jax.ShapeDtypeStruct((2, N, HALF), _f32),
  )(act, act, st, st, g_h, g_h, be_h, be_h, w_r, w_r, deg, deg)


def _pool_body(agg_ref, d0_ref, d1_ref, b_ref, batch_ref, pooled_ref,
               cnt_ref):
  y = agg_ref[0] * _dinv_of(d0_ref, d1_ref) + b_ref[0]
  bidx = batch_ref[0]
  p1h = (lax.broadcasted_iota(jnp.int32, (G, BN), 0) == bidx).astype(_f32)
  contrib = jnp.dot(p1h, y, preferred_element_type=_f32)
  cnt = jnp.dot(p1h, jnp.ones((BN, HALF), _f32), preferred_element_type=_f32)
  h = pl.program_id(0)
  i = pl.program_id(1)

  @pl.when(i == 0)
  def _():
    pooled_ref[0] = contrib

  @pl.when(i > 0)
  def _():
    pooled_ref[0] = pooled_ref[0] + contrib

  @pl.when((h == 0) & (i == 0))
  def _():
    cnt_ref[...] = cnt

  @pl.when((h == 0) & (i > 0))
  def _():
    cnt_ref[...] = cnt_ref[...] + cnt


def _tc_pool(agg, deg, bias_h, batch_r):
  grid = (2, NB)
  return pl.pallas_call(
      _pool_body,
      grid=grid,
      in_specs=[
          pl.BlockSpec((1, BN, HALF), lambda h, i: (h, i, 0)),
          pl.BlockSpec((1, BN, HALF), lambda h, i: (0, i, 0)),
          pl.BlockSpec((1, BN, HALF), lambda h, i: (1, i, 0)),
          pl.BlockSpec((1, 1, HALF), lambda h, i: (h, 0, 0)),
          pl.BlockSpec((1, 1, BN), lambda h, i: (i, 0, 0)),
      ],
      out_specs=[
          pl.BlockSpec((1, G, HALF), lambda h, i: (h, 0, 0)),
          pl.BlockSpec((G, HALF), lambda h, i: (0, 0)),
      ],
      out_shape=[
          jax.ShapeDtypeStruct((2, G, HALF), _f32),
          jax.ShapeDtypeStruct((G, HALF), _f32),
      ],
  )(agg, deg, deg, bias_h, batch_r)


def _head_body(p1_ref, c1_ref, p2_ref, c2_ref, wl1_ref, bl1_ref, wl2_ref,
               bl2_ref, o_ref):
  def embed(p_ref, c_ref):
    p = jnp.concatenate([p_ref[0], p_ref[1]], axis=1)
    c = jnp.maximum(c_ref[:, 0:1], 1.0)
    m = p / c
    t = jnp.dot(m, wl1_ref[...], preferred_element_type=_f32) + bl1_ref[...]
    t = jnp.maximum(t, 0.0)
    e = jnp.dot(t, wl2_ref[...], preferred_element_type=_f32) + bl2_ref[...]
    return jnp.maximum(e, 0.0)

  e1 = embed(p1_ref, c1_ref)
  e2 = embed(p2_ref, c2_ref)
  o_ref[...] = jnp.sum(jnp.abs(e1 - e2), axis=1, keepdims=True)


def _tc_head(p1, c1, p2, c2, wl1, bl1, wl2, bl2):
  return pl.pallas_call(
      _head_body,
      out_shape=jax.ShapeDtypeStruct((G, 1), _f32),
  )(p1, c1, p2, c2, wl1, bl1, wl2, bl2)


def _net_chain(x, src, dst, batch, Wc1, bc1h, wc2r, bc2h, wc3r, bc3h,
               g1h, be1h, g2h, be2h):
  joff = jnp.arange(2, dtype=jnp.int32)[:, None] * N
  src_jobs = (src[None, :] + joff).reshape(2, NS, NIB, IB, CW)
  dst_b = dst.reshape(NS, NIB, IB, CW)
  dst_d = dst.reshape(2, NS, NIBD, IB, CW)
  batch_r = batch.reshape(NB, 1, BN)

  deg = _sc_degrees(dst_d)

  h1 = _tc_conv1(x, Wc1, deg)
  agg1 = _sc_aggregate(h1.reshape(2 * N, HALF), src_jobs, dst_b)
  act1, st1 = _tc_epilogue(agg1.reshape(2, N, HALF), deg, bc1h)

  h2 = _tc_bn_matmul(act1, st1, g1h, be1h, wc2r, deg)
  agg2 = _sc_aggregate(h2.reshape(2 * N, HALF), src_jobs, dst_b)
  act2, st2 = _tc_epilogue(agg2.reshape(2, N, HALF), deg, bc2h)

  h3 = _tc_bn_matmul(act2, st2, g2h, be2h, wc3r, deg)
  agg3 = _sc_aggregate(h3.reshape(2 * N, HALF), src_jobs, dst_b)
  return _tc_pool(agg3.reshape(2, N, HALF), deg, bc3h, batch_r)


def kernel(x1, edge_index1, batch1, x2, edge_index2, batch2,
           Wc1, bc1, Wc2, bc2, Wc3, bc3, g1, be1, g2, be2, Wl1, bl1, Wl2,
           bl2):
  bc1h = bc1.reshape(2, 1, HALF)
  bc2h = bc2.reshape(2, 1, HALF)
  bc3h = bc3.reshape(2, 1, HALF)
  g1h = g1.reshape(2, 1, HALF)
  be1h = be1.reshape(2, 1, HALF)
  g2h = g2.reshape(2, 1, HALF)
  be2h = be2.reshape(2, 1, HALF)
  wc2r = Wc2.reshape(2, HALF, H)
  wc3r = Wc3.reshape(2, HALF, H)

  p1, c1 = _net_chain(x1, edge_index1[0], edge_index1[1], batch1,
                      Wc1, bc1h, wc2r, bc2h, wc3r, bc3h,
                      g1h, be1h, g2h, be2h)
  p2, c2 = _net_chain(x2, edge_index2[0], edge_index2[1], batch2,
                      Wc1, bc1h, wc2r, bc2h, wc3r, bc3h,
                      g1h, be1h, g2h, be2h)

  out = _tc_head(p1, c1, p2, c2, Wl1, bl1.reshape(1, H), Wl2,
                 bl2.reshape(1, D_OUT_))
  return out[:, 0]


# speedup vs baseline: 1.4546x; 1.0041x over previous
import jax
import jax.numpy as jnp
from jax import lax
from jax.experimental import pallas as pl
from jax.experimental.pallas import tpu as pltpu
from jax.experimental.pallas import tpu_sc as plsc

N = 10000
E = 320000
D_IN = 128
H = 256
HALF = 128
G = 64
D_OUT_ = 128

NS = 16
EPT = E // NS
CW = 80
NCH = EPT // CW
IB = 25
NIB = NCH // IB
NIBD = NIB // 2
NPT = 624
TAIL_OFF = NS * NPT
TAIL = N - TAIL_OFF

NBUF = 3
BN = 1000
NB = N // BN

_f32 = jnp.float32


def _sc_mesh():
  return plsc.VectorSubcoreMesh(core_axis_name="c", subcore_axis_name="s")


def _deg_body(dst_hbm, zeros_hbm, ones_hbm, out_hbm, idx_v, ones_v, acc_sh,
              sem_s, sem_i):
  c = lax.axis_index("c")
  s = lax.axis_index("s")
  off = s * NPT
  pltpu.sync_copy(zeros_hbm.at[pl.ds(off, NPT)], acc_sh.at[pl.ds(off, NPT)])

  @pl.when(s == NS - 1)
  def _():
    pltpu.sync_copy(zeros_hbm.at[pl.ds(TAIL_OFF, TAIL)],
                    acc_sh.at[pl.ds(TAIL_OFF, TAIL)])

  pltpu.sync_copy(ones_hbm, ones_v)
  plsc.subcore_barrier()
  pltpu.sync_copy(dst_hbm.at[c, s, 0], idx_v.at[0])

  @pl.loop(0, NIBD)
  def _(blk):
    ib = lax.rem(blk, 2)

    @pl.when(blk + 1 < NIBD)
    def _():
      pltpu.async_copy(dst_hbm.at[c, s, blk + 1], idx_v.at[1 - ib], sem_i)

    @pl.loop(0, IB)
    def _(k):
      pltpu.async_copy(ones_v, acc_sh.at[idx_v.at[ib, k]], sem_s, add=True)

    @pl.loop(0, IB)
    def _(k):
      pltpu.make_async_copy(ones_v, acc_sh.at[idx_v.at[ib, k]], sem_s).wait()

    @pl.when(blk + 1 < NIBD)
    def _():
      pltpu.make_async_copy(dst_hbm.at[c, s, 0], idx_v.at[1 - ib],
                            sem_i).wait()

  plsc.subcore_barrier()
  pltpu.sync_copy(acc_sh.at[pl.ds(off, NPT)], out_hbm.at[c, pl.ds(off, NPT)])

  @pl.when(s == NS - 1)
  def _():
    pltpu.sync_copy(acc_sh.at[pl.ds(TAIL_OFF, TAIL)],
                    out_hbm.at[c, pl.ds(TAIL_OFF, TAIL)])


def _sc_degrees(dst_d):
  zeros = jnp.zeros((N, HALF), _f32)
  ones = jnp.ones((CW, HALF), _f32)
  k = pl.kernel(
      _deg_body,
      out_type=jax.ShapeDtypeStruct((2, N, HALF), _f32),
      mesh=_sc_mesh(),
      scratch_types=[
          pltpu.VMEM((2, IB, CW), jnp.int32),
          pltpu.VMEM((CW, HALF), _f32),
          pltpu.VMEM_SHARED((N, HALF), _f32),
          pltpu.SemaphoreType.DMA,
          pltpu.SemaphoreType.DMA,
      ],
  )
  return k(dst_d, zeros, ones)


def _agg_body(h_hbm, src_hbm, dst_hbm, out_hbm, src_v, dst_v, rows_v, acc_sh,
              sem_g, sem_s, sem_i):
  c = lax.axis_index("c")
  s = lax.axis_index("s")
  off = s * NPT
  base = c * N

  def gather_start(ib, k, buf):
    pltpu.async_copy(h_hbm.at[src_v.at[ib, k]], rows_v.at[buf],
                     sem_g.at[buf])

  def gather_wait(ib, k, buf):
    pltpu.make_async_copy(h_hbm.at[src_v.at[ib, k]], rows_v.at[buf],
                          sem_g.at[buf]).wait()

  def scatter_start(ib, k, buf):
    pltpu.async_copy(rows_v.at[buf], acc_sh.at[dst_v.at[ib, k]], sem_s,
                     add=True)

  def scatter_wait(ib, k, buf):
    pltpu.make_async_copy(rows_v.at[buf], acc_sh.at[dst_v.at[ib, k]],
                          sem_s).wait()

  pltpu.sync_copy(h_hbm.at[pl.ds(base + off, NPT)],
                  acc_sh.at[pl.ds(off, NPT)])

  @pl.when(s == NS - 1)
  def _():
    pltpu.sync_copy(h_hbm.at[pl.ds(base + TAIL_OFF, TAIL)],
                    acc_sh.at[pl.ds(TAIL_OFF, TAIL)])

  plsc.subcore_barrier()
  pltpu.sync_copy(src_hbm.at[c, s, 0], src_v.at[0])
  pltpu.sync_copy(dst_hbm.at[s, 0], dst_v.at[0])
  gather_start(0, 0, 0)

  @pl.loop(0, NIB)
  def _(blk):
    ib = lax.rem(blk, 2)

    @pl.when(blk + 1 < NIB)
    def _():
      pltpu.async_copy(src_hbm.at[c, s, blk + 1], src_v.at[1 - ib], sem_i)
      pltpu.async_copy(dst_hbm.at[s, blk + 1], dst_v.at[1 - ib], sem_i)

    @pl.loop(0, IB)
    def _(k):
      buf = lax.rem(k, NBUF)
      nxt = lax.rem(k + 1, NBUF)

      @pl.when(k >= NBUF - 1)
      def _():
        scatter_wait(ib, k - (NBUF - 1), lax.rem(k + 1, NBUF))

      @pl.when(k + 1 < IB)
      def _():
        gather_start(ib, k + 1, nxt)

      gather_wait(ib, k, buf)
      scatter_start(ib, k, buf)

    for t in range(NBUF - 1):
      scatter_wait(ib, IB - (NBUF - 1) + t, (IB - (NBUF - 1) + t) % NBUF)

    @pl.when(blk + 1 < NIB)
    def _():
      pltpu.make_async_copy(src_hbm.at[c, s, 0], src_v.at[1 - ib],
                            sem_i).wait()
      pltpu.make_async_copy(dst_hbm.at[s, 0], dst_v.at[1 - ib],
                            sem_i).wait()
      gather_start(1 - ib, 0, 0)

  plsc.subcore_barrier()
  pltpu.sync_copy(acc_sh.at[pl.ds(off, NPT)],
                  out_hbm.at[pl.ds(base + off, NPT)])

  @pl.when(s == NS - 1)
  def _():
    pltpu.sync_copy(acc_sh.at[pl.ds(TAIL_OFF, TAIL)],
                    out_hbm.at[pl.ds(base + TAIL_OFF, TAIL)])


def _sc_aggregate(h_flat, src_jobs, dst_b):
  k = pl.kernel(
      _agg_body,
      out_type=jax.ShapeDtypeStruct((2 * N, HALF), _f32),
      mesh=_sc_mesh(),
      scratch_types=[
          pltpu.VMEM((2, IB, CW), jnp.int32),
          pltpu.VMEM((2, IB, CW), jnp.int32),
          pltpu.VMEM((NBUF, CW, HALF), _f32),
          pltpu.VMEM_SHARED((N, HALF), _f32),
          pltpu.SemaphoreType.DMA((NBUF,)),
          pltpu.SemaphoreType.DMA,
          pltpu.SemaphoreType.DMA,
      ],
  )
  return k(h_flat, src_jobs, dst_b)


def _dinv_of(d0_ref, d1_ref):
  return lax.rsqrt(d0_ref[0, :, 0:1] + d1_ref[0, :, 0:1] + 1.0)


def _a1_body(x_ref, w_ref, d0_ref, d1_ref, o_ref):
  h = jnp.dot(x_ref[...], w_ref[...], preferred_element_type=_f32)
  o_ref[0] = h * _dinv_of(d0_ref, d1_ref)


def _tc_conv1(x, w, deg):
  grid = (2, NB)
  return pl.pallas_call(
      _a1_body,
      grid=grid,
      in_specs=[
          pl.BlockSpec((BN, D_IN), lambda h, i: (i, 0)),
          pl.BlockSpec((D_IN, HALF), lambda h, i: (0, h)),
          pl.BlockSpec((1, BN, HALF), lambda h, i: (0, i, 0)),
          pl.BlockSpec((1, BN, HALF), lambda h, i: (1, i, 0)),
      ],
      out_specs=pl.BlockSpec((1, BN, HALF), lambda h, i: (h, i, 0)),
      out_shape=jax.ShapeDtypeStruct((2, N, HALF), _f32),
  )(x, w, deg, deg)


def _epi_body(agg_ref, d0_ref, d1_ref, b_ref, act_ref, st_ref):
  y = agg_ref[0] * _dinv_of(d0_ref, d1_ref) + b_ref[0]
  y = jnp.maximum(y, 0.0)
  act_ref[0] = y
  sums = jnp.sum(y, axis=0, keepdims=True)
  sq = jnp.sum(y * y, axis=0, keepdims=True)
  upd = jnp.concatenate([sums, sq], axis=0)
  i = pl.program_id(1)

  @pl.when(i == 0)
  def _():
    st_ref[0] = upd

  @pl.when(i > 0)
  def _():
    st_ref[0] = st_ref[0] + upd


def _tc_epilogue(agg, deg, bias_h):
  grid = (2, NB)
  return pl.pallas_call(
      _epi_body,
      grid=grid,
      in_specs=[
          pl.BlockSpec((1, BN, HALF), lambda h, i: (h, i, 0)),
          pl.BlockSpec((1, BN, HALF), lambda h, i: (0, i, 0)),
          pl.BlockSpec((1, BN, HALF), lambda h, i: (1, i, 0)),
          pl.BlockSpec((1, 1, HALF), lambda h, i: (h, 0, 0)),
      ],
      out_specs=[
          pl.BlockSpec((1, BN, HALF), lambda h, i: (h, i, 0)),
          pl.BlockSpec((1, 2, HALF), lambda h, i: (h, 0, 0)),
      ],
      out_shape=[
          jax.ShapeDtypeStruct((2, N, HALF), _f32),
          jax.ShapeDtypeStruct((2, 2, HALF), _f32),
      ],
  )(agg, deg, deg, bias_h)


def _bn_half(act, st, g, be):
  m = st[0:1, :] * (1.0 / N)
  v = st[1:2, :] * (1.0 / N) - m * m
  a = g * lax.rsqrt(v + 1e-5)
  c = be - m * a
  return act * a + c


def _amat_body(alo_ref, ahi_ref, stlo_ref, sthi_ref, glo_ref, ghi_ref,
               belo_ref, behi_ref, wlo_ref, whi_ref, d0_ref, d1_ref, o_ref):
  x0 = _bn_half(alo_ref[0], stlo_ref[0], glo_ref[0], belo_ref[0])
  x1 = _bn_half(ahi_ref[0], sthi_ref[0], ghi_ref[0], behi_ref[0])
  h = (jnp.dot(x0, wlo_ref[0], preferred_element_type=_f32) +
       jnp.dot(x1, whi_ref[0], preferred_element_type=_f32))
  o_ref[0] = h * _dinv_of(d0_ref, d1_ref)


def _tc_bn_matmul(act, st, g_h, be_h, w_r, deg):
  grid = (2, NB)
  act_spec = lambda k: pl.BlockSpec((1, BN, HALF),
                                    lambda h, i, _k=k: (_k, i, 0))
  st_spec = lambda k: pl.BlockSpec((1, 2, HALF),
                                   lambda h, i, _k=k: (_k, 0, 0))
  vec_spec = lambda k: pl.BlockSpec((1, 1, HALF),
                                    lambda h, i, _k=k: (_k, 0, 0))
  w_spec = lambda k: pl.BlockSpec((1, HALF, HALF),
                                  lambda h, i, _k=k: (_k, 0, h))
  deg_spec = lambda k: pl.BlockSpec((1, BN, HALF),
                                    lambda h, i, _k=k: (_k, i, 0))
  return pl.pallas_call(
      _amat_body,
      grid=grid,
      in_specs=[
          act_spec(0), act_spec(1), st_spec(0), st_spec(1),
          vec_spec(0), vec_spec(1), vec_spec(0), vec_spec(1),
          w_spec(0), w_spec(1), deg_spec(0), deg_spec(1),
      ],
      out_specs=pl.BlockSpec((1, BN, HALF), lambda h, i: (h, i, 0)),
      out_shape=jax.ShapeDtypeStruct((2, N, HALF), _f32),
  )(act, act, st, st, g_h, g_h, be_h, be_h, w_r, w_r, deg, deg)


def _pool_body(agg_ref, d0_ref, d1_ref, b_ref, batch_ref, pooled_ref,
               cnt_ref):
  y = agg_ref[0] * _dinv_of(d0_ref, d1_ref) + b_ref[0]
  bidx = batch_ref[0]
  p1h = (lax.broadcasted_iota(jnp.int32, (G, BN), 0) == bidx).astype(_f32)
  contrib = jnp.dot(p1h, y, preferred_element_type=_f32)
  cnt = jnp.dot(p1h, jnp.ones((BN, HALF), _f32), preferred_element_type=_f32)
  h = pl.program_id(0)
  i = pl.program_id(1)

  @pl.when(i == 0)
  def _():
    pooled_ref[0] = contrib

  @pl.when(i > 0)
  def _():
    pooled_ref[0] = pooled_ref[0] + contrib

  @pl.when((h == 0) & (i == 0))
  def _():
    cnt_ref[...] = cnt

  @pl.when((h == 0) & (i > 0))
  def _():
    cnt_ref[...] = cnt_ref[...] + cnt


def _tc_pool(agg, deg, bias_h, batch_r):
  grid = (2, NB)
  return pl.pallas_call(
      _pool_body,
      grid=grid,
      in_specs=[
          pl.BlockSpec((1, BN, HALF), lambda h, i: (h, i, 0)),
          pl.BlockSpec((1, BN, HALF), lambda h, i: (0, i, 0)),
          pl.BlockSpec((1, BN, HALF), lambda h, i: (1, i, 0)),
          pl.BlockSpec((1, 1, HALF), lambda h, i: (h, 0, 0)),
          pl.BlockSpec((1, 1, BN), lambda h, i: (i, 0, 0)),
      ],
      out_specs=[
          pl.BlockSpec((1, G, HALF), lambda h, i: (h, 0, 0)),
          pl.BlockSpec((G, HALF), lambda h, i: (0, 0)),
      ],
      out_shape=[
          jax.ShapeDtypeStruct((2, G, HALF), _f32),
          jax.ShapeDtypeStruct((G, HALF), _f32),
      ],
  )(agg, deg, deg, bias_h, batch_r)


def _head_body(p1_ref, c1_ref, p2_ref, c2_ref, wl1_ref, bl1_ref, wl2_ref,
               bl2_ref, o_ref):
  def embed(p_ref, c_ref):
    p = jnp.concatenate([p_ref[0], p_ref[1]], axis=1)
    c = jnp.maximum(c_ref[:, 0:1], 1.0)
    m = p / c
    t = jnp.dot(m, wl1_ref[...], preferred_element_type=_f32) + bl1_ref[...]
    t = jnp.maximum(t, 0.0)
    e = jnp.dot(t, wl2_ref[...], preferred_element_type=_f32) + bl2_ref[...]
    return jnp.maximum(e, 0.0)

  e1 = embed(p1_ref, c1_ref)
  e2 = embed(p2_ref, c2_ref)
  o_ref[...] = jnp.sum(jnp.abs(e1 - e2), axis=1, keepdims=True)


def _tc_head(p1, c1, p2, c2, wl1, bl1, wl2, bl2):
  return pl.pallas_call(
      _head_body,
      out_shape=jax.ShapeDtypeStruct((G, 1), _f32),
  )(p1, c1, p2, c2, wl1, bl1, wl2, bl2)


def _net_chain(x, src, dst, batch, Wc1, bc1h, wc2r, bc2h, wc3r, bc3h,
               g1h, be1h, g2h, be2h):
  joff = jnp.arange(2, dtype=jnp.int32)[:, None] * N
  src_jobs = (src[None, :] + joff).reshape(2, NS, NIB, IB, CW)
  dst_b = dst.reshape(NS, NIB, IB, CW)
  dst_d = dst.reshape(2, NS, NIBD, IB, CW)
  batch_r = batch.reshape(NB, 1, BN)

  deg = _sc_degrees(dst_d)

  h1 = _tc_conv1(x, Wc1, deg)
  agg1 = _sc_aggregate(h1.reshape(2 * N, HALF), src_jobs, dst_b)
  act1, st1 = _tc_epilogue(agg1.reshape(2, N, HALF), deg, bc1h)

  h2 = _tc_bn_matmul(act1, st1, g1h, be1h, wc2r, deg)
  agg2 = _sc_aggregate(h2.reshape(2 * N, HALF), src_jobs, dst_b)
  act2, st2 = _tc_epilogue(agg2.reshape(2, N, HALF), deg, bc2h)

  h3 = _tc_bn_matmul(act2, st2, g2h, be2h, wc3r, deg)
  agg3 = _sc_aggregate(h3.reshape(2 * N, HALF), src_jobs, dst_b)
  return _tc_pool(agg3.reshape(2, N, HALF), deg, bc3h, batch_r)


def kernel(x1, edge_index1, batch1, x2, edge_index2, batch2,
           Wc1, bc1, Wc2, bc2, Wc3, bc3, g1, be1, g2, be2, Wl1, bl1, Wl2,
           bl2):
  bc1h = bc1.reshape(2, 1, HALF)
  bc2h = bc2.reshape(2, 1, HALF)
  bc3h = bc3.reshape(2, 1, HALF)
  g1h = g1.reshape(2, 1, HALF)
  be1h = be1.reshape(2, 1, HALF)
  g2h = g2.reshape(2, 1, HALF)
  be2h = be2.reshape(2, 1, HALF)
  wc2r = Wc2.reshape(2, HALF, H)
  wc3r = Wc3.reshape(2, HALF, H)

  p1, c1 = _net_chain(x1, edge_index1[0], edge_index1[1], batch1,
                      Wc1, bc1h, wc2r, bc2h, wc3r, bc3h,
                      g1h, be1h, g2h, be2h)
  p2, c2 = _net_chain(x2, edge_index2[0], edge_index2[1], batch2,
                      Wc1, bc1h, wc2r, bc2h, wc3r, bc3h,
                      g1h, be1h, g2h, be2h)

  out = _tc_head(p1, c1, p2, c2, Wl1, bl1.reshape(1, H), Wl2,
                 bl2.reshape(1, D_OUT_))
  return out[:, 0]


# speedup vs baseline: 1.5004x; 1.0315x over previous
import jax
import jax.numpy as jnp
from jax import lax
from jax.experimental import pallas as pl
from jax.experimental.pallas import tpu as pltpu
from jax.experimental.pallas import tpu_sc as plsc

N = 10000
E = 320000
D_IN = 128
H = 256
HALF = 128
G = 64
D_OUT_ = 128

NS = 16
EPT = E // NS
CW = 80
NCH = EPT // CW
IB = 25
NIB = NCH // IB
NIBD = NIB // 2
NPT = 624
TAIL_OFF = NS * NPT
TAIL = N - TAIL_OFF

NBUF = 3
BN = 1000
NB = N // BN

_f32 = jnp.float32


def _sc_mesh():
  return plsc.VectorSubcoreMesh(core_axis_name="c", subcore_axis_name="s")


def _deg_body(dst_hbm, zeros_hbm, ones_hbm, out_hbm, idx_v, ones_v, acc_sh,
              sem_s, sem_i):
  c = lax.axis_index("c")
  s = lax.axis_index("s")
  off = s * NPT
  pltpu.sync_copy(zeros_hbm.at[pl.ds(off, NPT)], acc_sh.at[pl.ds(off, NPT)])

  @pl.when(s == NS - 1)
  def _():
    pltpu.sync_copy(zeros_hbm.at[pl.ds(TAIL_OFF, TAIL)],
                    acc_sh.at[pl.ds(TAIL_OFF, TAIL)])

  pltpu.sync_copy(ones_hbm, ones_v)
  plsc.subcore_barrier()
  pltpu.sync_copy(dst_hbm.at[c, s, 0], idx_v.at[0])

  @pl.loop(0, NIBD)
  def _(blk):
    ib = lax.rem(blk, 2)

    @pl.when(blk + 1 < NIBD)
    def _():
      pltpu.async_copy(dst_hbm.at[c, s, blk + 1], idx_v.at[1 - ib], sem_i)

    @pl.loop(0, IB)
    def _(k):
      pltpu.async_copy(ones_v, acc_sh.at[idx_v.at[ib, k]], sem_s, add=True)

    @pl.loop(0, IB)
    def _(k):
      pltpu.make_async_copy(ones_v, acc_sh.at[idx_v.at[ib, k]], sem_s).wait()

    @pl.when(blk + 1 < NIBD)
    def _():
      pltpu.make_async_copy(dst_hbm.at[c, s, 0], idx_v.at[1 - ib],
                            sem_i).wait()

  plsc.subcore_barrier()
  pltpu.sync_copy(acc_sh.at[pl.ds(off, NPT)], out_hbm.at[c, pl.ds(off, NPT)])

  @pl.when(s == NS - 1)
  def _():
    pltpu.sync_copy(acc_sh.at[pl.ds(TAIL_OFF, TAIL)],
                    out_hbm.at[c, pl.ds(TAIL_OFF, TAIL)])


def _sc_degrees(dst_d):
  zeros = jnp.zeros((N, HALF), _f32)
  ones = jnp.ones((CW, HALF), _f32)
  k = pl.kernel(
      _deg_body,
      out_type=jax.ShapeDtypeStruct((2, N, HALF), _f32),
      mesh=_sc_mesh(),
      scratch_types=[
          pltpu.VMEM((2, IB, CW), jnp.int32),
          pltpu.VMEM((CW, HALF), _f32),
          pltpu.VMEM_SHARED((N, HALF), _f32),
          pltpu.SemaphoreType.DMA,
          pltpu.SemaphoreType.DMA,
      ],
  )
  return k(dst_d, zeros, ones)


def _agg_body(h_hbm, src_hbm, dst_hbm, out_hbm, src_v, dst_v, rows_v, acc_sh,
              sem_g, sem_s, sem_i):
  c = lax.axis_index("c")
  s = lax.axis_index("s")
  off = s * NPT
  base = c * N

  def gather_start(ib, k, buf):
    pltpu.async_copy(h_hbm.at[src_v.at[ib, k]], rows_v.at[buf],
                     sem_g.at[buf])

  def gather_wait(ib, k, buf):
    pltpu.make_async_copy(h_hbm.at[src_v.at[ib, k]], rows_v.at[buf],
                          sem_g.at[buf]).wait()

  def scatter_start(ib, k, buf):
    pltpu.async_copy(rows_v.at[buf], acc_sh.at[dst_v.at[ib, k]], sem_s,
                     add=True)

  def scatter_wait(ib, k, buf):
    pltpu.make_async_copy(rows_v.at[buf], acc_sh.at[dst_v.at[ib, k]],
                          sem_s).wait()

  pltpu.sync_copy(h_hbm.at[pl.ds(base + off, NPT)],
                  acc_sh.at[pl.ds(off, NPT)])

  @pl.when(s == NS - 1)
  def _():
    pltpu.sync_copy(h_hbm.at[pl.ds(base + TAIL_OFF, TAIL)],
                    acc_sh.at[pl.ds(TAIL_OFF, TAIL)])

  plsc.subcore_barrier()
  pltpu.sync_copy(src_hbm.at[c, s, 0], src_v.at[0])
  pltpu.sync_copy(dst_hbm.at[s, 0], dst_v.at[0])
  gather_start(0, 0, 0)

  @pl.loop(0, NCH)
  def _(g):
    blk = lax.div(g, IB)
    kk = lax.rem(g, IB)
    ib = lax.rem(blk, 2)

    @pl.when(g >= NBUF - 1)
    def _():
      g2 = g - (NBUF - 1)
      scatter_wait(lax.rem(lax.div(g2, IB), 2), lax.rem(g2, IB),
                   lax.rem(g2, NBUF))

    @pl.when((kk == 1) & (blk + 1 < NIB))
    def _():
      pltpu.async_copy(src_hbm.at[c, s, blk + 1], src_v.at[1 - ib], sem_i)
      pltpu.async_copy(dst_hbm.at[s, blk + 1], dst_v.at[1 - ib], sem_i)

    @pl.when((kk == IB - 1) & (blk + 1 < NIB))
    def _():
      pltpu.make_async_copy(src_hbm.at[c, s, 0], src_v.at[1 - ib],
                            sem_i).wait()
      pltpu.make_async_copy(dst_hbm.at[s, 0], dst_v.at[1 - ib],
                            sem_i).wait()

    @pl.when(g + 1 < NCH)
    def _():
      g2 = g + 1
      gather_start(lax.rem(lax.div(g2, IB), 2), lax.rem(g2, IB),
                   lax.rem(g2, NBUF))

    gather_wait(ib, kk, lax.rem(g, NBUF))
    scatter_start(ib, kk, lax.rem(g, NBUF))

  for t in range(NBUF - 1):
    g2 = NCH - (NBUF - 1) + t
    scatter_wait((g2 // IB) % 2, g2 % IB, g2 % NBUF)

  plsc.subcore_barrier()
  pltpu.sync_copy(acc_sh.at[pl.ds(off, NPT)],
                  out_hbm.at[pl.ds(base + off, NPT)])

  @pl.when(s == NS - 1)
  def _():
    pltpu.sync_copy(acc_sh.at[pl.ds(TAIL_OFF, TAIL)],
                    out_hbm.at[pl.ds(base + TAIL_OFF, TAIL)])


def _sc_aggregate(h_flat, src_jobs, dst_b):
  k = pl.kernel(
      _agg_body,
      out_type=jax.ShapeDtypeStruct((2 * N, HALF), _f32),
      mesh=_sc_mesh(),
      scratch_types=[
          pltpu.VMEM((2, IB, CW), jnp.int32),
          pltpu.VMEM((2, IB, CW), jnp.int32),
          pltpu.VMEM((NBUF, CW, HALF), _f32),
          pltpu.VMEM_SHARED((N, HALF), _f32),
          pltpu.SemaphoreType.DMA((NBUF,)),
          pltpu.SemaphoreType.DMA,
          pltpu.SemaphoreType.DMA,
      ],
  )
  return k(h_flat, src_jobs, dst_b)


def _dinv_of(d0_ref, d1_ref):
  return lax.rsqrt(d0_ref[0, :, 0:1] + d1_ref[0, :, 0:1] + 1.0)


def _a1_body(x_ref, w_ref, d0_ref, d1_ref, o_ref):
  h = jnp.dot(x_ref[...], w_ref[...], preferred_element_type=_f32)
  o_ref[0] = h * _dinv_of(d0_ref, d1_ref)


def _tc_conv1(x, w, deg):
  grid = (2, NB)
  return pl.pallas_call(
      _a1_body,
      grid=grid,
      in_specs=[
          pl.BlockSpec((BN, D_IN), lambda h, i: (i, 0)),
          pl.BlockSpec((D_IN, HALF), lambda h, i: (0, h)),
          pl.BlockSpec((1, BN, HALF), lambda h, i: (0, i, 0)),
          pl.BlockSpec((1, BN, HALF), lambda h, i: (1, i, 0)),
      ],
      out_specs=pl.BlockSpec((1, BN, HALF), lambda h, i: (h, i, 0)),
      out_shape=jax.ShapeDtypeStruct((2, N, HALF), _f32),
  )(x, w, deg, deg)


def _epi_body(agg_ref, d0_ref, d1_ref, b_ref, act_ref, st_ref):
  y = agg_ref[0] * _dinv_of(d0_ref, d1_ref) + b_ref[0]
  y = jnp.maximum(y, 0.0)
  act_ref[0] = y
  sums = jnp.sum(y, axis=0, keepdims=True)
  sq = jnp.sum(y * y, axis=0, keepdims=True)
  upd = jnp.concatenate([sums, sq], axis=0)
  i = pl.program_id(1)

  @pl.when(i == 0)
  def _():
    st_ref[0] = upd

  @pl.when(i > 0)
  def _():
    st_ref[0] = st_ref[0] + upd


def _tc_epilogue(agg, deg, bias_h):
  grid = (2, NB)
  return pl.pallas_call(
      _epi_body,
      grid=grid,
      in_specs=[
          pl.BlockSpec((1, BN, HALF), lambda h, i: (h, i, 0)),
          pl.BlockSpec((1, BN, HALF), lambda h, i: (0, i, 0)),
          pl.BlockSpec((1, BN, HALF), lambda h, i: (1, i, 0)),
          pl.BlockSpec((1, 1, HALF), lambda h, i: (h, 0, 0)),
      ],
      out_specs=[
          pl.BlockSpec((1, BN, HALF), lambda h, i: (h, i, 0)),
          pl.BlockSpec((1, 2, HALF), lambda h, i: (h, 0, 0)),
      ],
      out_shape=[
          jax.ShapeDtypeStruct((2, N, HALF), _f32),
          jax.ShapeDtypeStruct((2, 2, HALF), _f32),
      ],
  )(agg, deg, deg, bias_h)


def _bn_half(act, st, g, be):
  m = st[0:1, :] * (1.0 / N)
  v = st[1:2, :] * (1.0 / N) - m * m
  a = g * lax.rsqrt(v + 1e-5)
  c = be - m * a
  return act * a + c


def _amat_body(alo_ref, ahi_ref, stlo_ref, sthi_ref, glo_ref, ghi_ref,
               belo_ref, behi_ref, wlo_ref, whi_ref, d0_ref, d1_ref, o_ref):
  x0 = _bn_half(alo_ref[0], stlo_ref[0], glo_ref[0], belo_ref[0])
  x1 = _bn_half(ahi_ref[0], sthi_ref[0], ghi_ref[0], behi_ref[0])
  h = (jnp.dot(x0, wlo_ref[0], preferred_element_type=_f32) +
       jnp.dot(x1, whi_ref[0], preferred_element_type=_f32))
  o_ref[0] = h * _dinv_of(d0_ref, d1_ref)


def _tc_bn_matmul(act, st, g_h, be_h, w_r, deg):
  grid = (2, NB)
  act_spec = lambda k: pl.BlockSpec((1, BN, HALF),
                                    lambda h, i, _k=k: (_k, i, 0))
  st_spec = lambda k: pl.BlockSpec((1, 2, HALF),
                                   lambda h, i, _k=k: (_k, 0, 0))
  vec_spec = lambda k: pl.BlockSpec((1, 1, HALF),
                                    lambda h, i, _k=k: (_k, 0, 0))
  w_spec = lambda k: pl.BlockSpec((1, HALF, HALF),
                                  lambda h, i, _k=k: (_k, 0, h))
  deg_spec = lambda k: pl.BlockSpec((1, BN, HALF),
                                    lambda h, i, _k=k: (_k, i, 0))
  return pl.pallas_call(
      _amat_body,
      grid=grid,
      in_specs=[
          act_spec(0), act_spec(1), st_spec(0), st_spec(1),
          vec_spec(0), vec_spec(1), vec_spec(0), vec_spec(1),
          w_spec(0), w_spec(1), deg_spec(0), deg_spec(1),
      ],
      out_specs=pl.BlockSpec((1, BN, HALF), lambda h, i: (h, i, 0)),
      out_shape=jax.ShapeDtypeStruct((2, N, HALF), _f32),
  )(act, act, st, st, g_h, g_h, be_h, be_h, w_r, w_r, deg, deg)


def _pool_body(agg_ref, d0_ref, d1_ref, b_ref, batch_ref, pooled_ref,
               cnt_ref):
  y = agg_ref[0] * _dinv_of(d0_ref, d1_ref) + b_ref[0]
  bidx = batch_ref[0]
  p1h = (lax.broadcasted_iota(jnp.int32, (G, BN), 0) == bidx).astype(_f32)
  contrib = jnp.dot(p1h, y, preferred_element_type=_f32)
  cnt = jnp.dot(p1h, jnp.ones((BN, HALF), _f32), preferred_element_type=_f32)
  h = pl.program_id(0)
  i = pl.program_id(1)

  @pl.when(i == 0)
  def _():
    pooled_ref[0] = contrib

  @pl.when(i > 0)
  def _():
    pooled_ref[0] = pooled_ref[0] + contrib

  @pl.when((h == 0) & (i == 0))
  def _():
    cnt_ref[...] = cnt

  @pl.when((h == 0) & (i > 0))
  def _():
    cnt_ref[...] = cnt_ref[...] + cnt


def _tc_pool(agg, deg, bias_h, batch_r):
  grid = (2, NB)
  return pl.pallas_call(
      _pool_body,
      grid=grid,
      in_specs=[
          pl.BlockSpec((1, BN, HALF), lambda h, i: (h, i, 0)),
          pl.BlockSpec((1, BN, HALF), lambda h, i: (0, i, 0)),
          pl.BlockSpec((1, BN, HALF), lambda h, i: (1, i, 0)),
          pl.BlockSpec((1, 1, HALF), lambda h, i: (h, 0, 0)),
          pl.BlockSpec((1, 1, BN), lambda h, i: (i, 0, 0)),
      ],
      out_specs=[
          pl.BlockSpec((1, G, HALF), lambda h, i: (h, 0, 0)),
          pl.BlockSpec((G, HALF), lambda h, i: (0, 0)),
      ],
      out_shape=[
          jax.ShapeDtypeStruct((2, G, HALF), _f32),
          jax.ShapeDtypeStruct((G, HALF), _f32),
      ],
  )(agg, deg, deg, bias_h, batch_r)


def _head_body(p1_ref, c1_ref, p2_ref, c2_ref, wl1_ref, bl1_ref, wl2_ref,
               bl2_ref, o_ref):
  def embed(p_ref, c_ref):
    p = jnp.concatenate([p_ref[0], p_ref[1]], axis=1)
    c = jnp.maximum(c_ref[:, 0:1], 1.0)
    m = p / c
    t = jnp.dot(m, wl1_ref[...], preferred_element_type=_f32) + bl1_ref[...]
    t = jnp.maximum(t, 0.0)
    e = jnp.dot(t, wl2_ref[...], preferred_element_type=_f32) + bl2_ref[...]
    return jnp.maximum(e, 0.0)

  e1 = embed(p1_ref, c1_ref)
  e2 = embed(p2_ref, c2_ref)
  o_ref[...] = jnp.sum(jnp.abs(e1 - e2), axis=1, keepdims=True)


def _tc_head(p1, c1, p2, c2, wl1, bl1, wl2, bl2):
  return pl.pallas_call(
      _head_body,
      out_shape=jax.ShapeDtypeStruct((G, 1), _f32),
  )(p1, c1, p2, c2, wl1, bl1, wl2, bl2)


def _net_chain(x, src, dst, batch, Wc1, bc1h, wc2r, bc2h, wc3r, bc3h,
               g1h, be1h, g2h, be2h):
  joff = jnp.arange(2, dtype=jnp.int32)[:, None] * N
  src_jobs = (src[None, :] + joff).reshape(2, NS, NIB, IB, CW)
  dst_b = dst.reshape(NS, NIB, IB, CW)
  dst_d = dst.reshape(2, NS, NIBD, IB, CW)
  batch_r = batch.reshape(NB, 1, BN)

  deg = _sc_degrees(dst_d)

  h1 = _tc_conv1(x, Wc1, deg)
  agg1 = _sc_aggregate(h1.reshape(2 * N, HALF), src_jobs, dst_b)
  act1, st1 = _tc_epilogue(agg1.reshape(2, N, HALF), deg, bc1h)

  h2 = _tc_bn_matmul(act1, st1, g1h, be1h, wc2r, deg)
  agg2 = _sc_aggregate(h2.reshape(2 * N, HALF), src_jobs, dst_b)
  act2, st2 = _tc_epilogue(agg2.reshape(2, N, HALF), deg, bc2h)

  h3 = _tc_bn_matmul(act2, st2, g2h, be2h, wc3r, deg)
  agg3 = _sc_aggregate(h3.reshape(2 * N, HALF), src_jobs, dst_b)
  return _tc_pool(agg3.reshape(2, N, HALF), deg, bc3h, batch_r)


def kernel(x1, edge_index1, batch1, x2, edge_index2, batch2,
           Wc1, bc1, Wc2, bc2, Wc3, bc3, g1, be1, g2, be2, Wl1, bl1, Wl2,
           bl2):
  bc1h = bc1.reshape(2, 1, HALF)
  bc2h = bc2.reshape(2, 1, HALF)
  bc3h = bc3.reshape(2, 1, HALF)
  g1h = g1.reshape(2, 1, HALF)
  be1h = be1.reshape(2, 1, HALF)
  g2h = g2.reshape(2, 1, HALF)
  be2h = be2.reshape(2, 1, HALF)
  wc2r = Wc2.reshape(2, HALF, H)
  wc3r = Wc3.reshape(2, HALF, H)

  p1, c1 = _net_chain(x1, edge_index1[0], edge_index1[1], batch1,
                      Wc1, bc1h, wc2r, bc2h, wc3r, bc3h,
                      g1h, be1h, g2h, be2h)
  p2, c2 = _net_chain(x2, edge_index2[0], edge_index2[1], batch2,
                      Wc1, bc1h, wc2r, bc2h, wc3r, bc3h,
                      g1h, be1h, g2h, be2h)

  out = _tc_head(p1, c1, p2, c2, Wl1, bl1.reshape(1, H), Wl2,
                 bl2.reshape(1, D_OUT_))
  return out[:, 0]
